# Initial kernel scaffold; baseline (speedup 1.0000x reference)
import functools

import jax
import jax.numpy as jnp
from jax import lax
from jax.experimental import pallas as pl
from jax.experimental.pallas import tpu as pltpu
from jax.experimental.pallas import tpu_sc as plsc

N_TOK = 2048
D_IN = 4096
D_OUT = 4096
FAN_IN = 128

NC = 2
NS = 16
NW = NC * NS
LANES = 16

NCOL = 16
PASSES = D_OUT // (NW * NCOL)
BLK_WORDS = NCOL * D_IN


def _densify_body(wt_in_hbm, mt_in_hbm, dense_hbm, blk, wv, mv):
    wid = lax.axis_index("s") * NC + lax.axis_index("c")

    lane = lax.iota(jnp.int32, LANES)
    base = lane * D_IN
    zeros = jnp.zeros((LANES,), jnp.float32)

    def one_pass(p, carry):
        j0 = (p * NW + wid) * NCOL

        pltpu.sync_copy(wt_in_hbm.at[:, pl.ds(j0, NCOL)], wv)
        pltpu.sync_copy(mt_in_hbm.at[:, pl.ds(j0, NCOL)], mv)

        def zero_step(i, c):
            off = i * (LANES * 8)
            for u in range(8):
                blk[pl.ds(off + u * LANES, LANES)] = zeros
            return c

        lax.fori_loop(0, BLK_WORDS // (LANES * 8), zero_step, 0)

        def scat_step(k, c):
            idx = base + mv[k, :]
            plsc.addupdate_scatter(blk, (idx,), wv[k, :])
            return c

        lax.fori_loop(0, FAN_IN, scat_step, 0)

        pltpu.sync_copy(blk, dense_hbm.at[pl.ds(j0 * D_IN, BLK_WORDS)])
        return carry

    lax.fori_loop(0, PASSES, one_pass, 0)


def _densify(condensed_weight, input_mask):
    wt_in = condensed_weight.T
    mt_in = input_mask.T
    mesh = plsc.VectorSubcoreMesh(core_axis_name="c", subcore_axis_name="s")
    dense_flat = pl.kernel(
        _densify_body,
        out_type=jax.ShapeDtypeStruct((D_OUT * D_IN,), jnp.float32),
        mesh=mesh,
        scratch_types=[
            pltpu.VMEM((BLK_WORDS,), jnp.float32),
            pltpu.VMEM((FAN_IN, NCOL), jnp.float32),
            pltpu.VMEM((FAN_IN, NCOL), jnp.int32),
        ],
    )(wt_in, mt_in)
    return dense_flat.reshape(D_OUT, D_IN)


M_BLK = 512
N_BLK = 512


def _matmul_body(x_ref, w_ref, b_ref, o_ref):
    acc = lax.dot_general(
        x_ref[...],
        w_ref[...],
        dimension_numbers=(((1,), (1,)), ((), ())),
        preferred_element_type=jnp.float32,
        precision=lax.Precision.HIGHEST,
    )
    o_ref[...] = acc + b_ref[...][None, :]


def _matmul(x, wt, bias):
    grid = (N_TOK // M_BLK, D_OUT // N_BLK)
    return pl.pallas_call(
        _matmul_body,
        grid=grid,
        in_specs=[
            pl.BlockSpec((M_BLK, D_IN), lambda m, n: (m, 0)),
            pl.BlockSpec((N_BLK, D_IN), lambda m, n: (n, 0)),
            pl.BlockSpec((N_BLK,), lambda m, n: (n,)),
        ],
        out_specs=pl.BlockSpec((M_BLK, N_BLK), lambda m, n: (m, n)),
        out_shape=jax.ShapeDtypeStruct((N_TOK, D_OUT), jnp.float32),
    )(x, wt, bias)


@jax.jit
def kernel(input, condensed_weight, input_mask, bias):
    wt = _densify(condensed_weight, input_mask)
    return _matmul(input, wt, bias)


# speedup vs baseline: 8.8704x; 8.8704x over previous
import functools

import jax
import jax.numpy as jnp
from jax import lax
from jax.experimental import pallas as pl
from jax.experimental.pallas import tpu as pltpu
from jax.experimental.pallas import tpu_sc as plsc

N_TOK = 2048
D_IN = 4096
D_OUT = 4096
FAN_IN = 128

NC = 2
NS = 16
NW = NC * NS
LANES = 16

NCOL = 16
JCHUNK = D_OUT // NW
SUBBLKS = JCHUNK // NCOL
BLK_WORDS = NCOL * D_IN


def _densify_body(wt_in_hbm, mt_in_hbm, dense_hbm, blk, wv, mv):
    wid = lax.axis_index("s") * NC + lax.axis_index("c")
    J0 = wid * JCHUNK

    lane = lax.iota(jnp.int32, LANES)
    base = lane * D_IN
    zeros = jnp.zeros((LANES,), jnp.float32)

    pltpu.sync_copy(wt_in_hbm.at[:, pl.ds(J0, JCHUNK)], wv)
    pltpu.sync_copy(mt_in_hbm.at[:, pl.ds(J0, JCHUNK)], mv)

    def one_sub(sb, carry):
        c0 = sb * NCOL

        def zero_step(i, c):
            off = i * (LANES * 8)
            for u in range(8):
                blk[pl.ds(off + u * LANES, LANES)] = zeros
            return c

        lax.fori_loop(0, BLK_WORDS // (LANES * 8), zero_step, 0)

        def scat_step(k, c):
            idx = base + mv[k, pl.ds(c0, NCOL)]
            plsc.addupdate_scatter(blk, (idx,), wv[k, pl.ds(c0, NCOL)])
            return c

        lax.fori_loop(0, FAN_IN, scat_step, 0)

        pltpu.sync_copy(blk, dense_hbm.at[pl.ds((J0 + c0) * D_IN, BLK_WORDS)])
        return carry

    lax.fori_loop(0, SUBBLKS, one_sub, 0)


def _densify(condensed_weight, input_mask):
    wt_in = condensed_weight.T
    mt_in = input_mask.T
    mesh = plsc.VectorSubcoreMesh(core_axis_name="c", subcore_axis_name="s")
    dense_flat = pl.kernel(
        _densify_body,
        out_type=jax.ShapeDtypeStruct((D_OUT * D_IN,), jnp.float32),
        mesh=mesh,
        compiler_params=pltpu.CompilerParams(needs_layout_passes=False),
        scratch_types=[
            pltpu.VMEM((BLK_WORDS,), jnp.float32),
            pltpu.VMEM((FAN_IN, JCHUNK), jnp.float32),
            pltpu.VMEM((FAN_IN, JCHUNK), jnp.int32),
        ],
    )(wt_in, mt_in)
    return dense_flat.reshape(D_OUT, D_IN)


M_BLK = 512
N_BLK = 512


def _matmul_body(x_ref, w_ref, b_ref, o_ref):
    acc = lax.dot_general(
        x_ref[...],
        w_ref[...],
        dimension_numbers=(((1,), (1,)), ((), ())),
        preferred_element_type=jnp.float32,
        precision=lax.Precision.HIGHEST,
    )
    o_ref[...] = acc + b_ref[...][None, :]


def _matmul(x, wt, bias):
    grid = (N_TOK // M_BLK, D_OUT // N_BLK)
    return pl.pallas_call(
        _matmul_body,
        grid=grid,
        in_specs=[
            pl.BlockSpec((M_BLK, D_IN), lambda m, n: (m, 0)),
            pl.BlockSpec((N_BLK, D_IN), lambda m, n: (n, 0)),
            pl.BlockSpec((N_BLK,), lambda m, n: (n,)),
        ],
        out_specs=pl.BlockSpec((M_BLK, N_BLK), lambda m, n: (m, n)),
        out_shape=jax.ShapeDtypeStruct((N_TOK, D_OUT), jnp.float32),
    )(x, wt, bias)


@jax.jit
def kernel(input, condensed_weight, input_mask, bias):
    wt = _densify(condensed_weight, input_mask)
    return _matmul(input, wt, bias)


# speedup vs baseline: 19.7573x; 2.2273x over previous
import functools

import jax
import jax.numpy as jnp
from jax import lax
from jax.experimental import pallas as pl
from jax.experimental.pallas import tpu as pltpu
from jax.experimental.pallas import tpu_sc as plsc

N_TOK = 2048
D_IN = 4096
D_OUT = 4096
FAN_IN = 128

NC = 2
NS = 16
NW = NC * NS
LANES = 16

NCOL = 16
JCHUNK = D_OUT // NW
SUBBLKS = JCHUNK // NCOL
BLK_WORDS = NCOL * D_IN


def _densify_body(wt_in_hbm, mt_in_hbm, dense_hbm, blk, wv, mv):
    wid = lax.axis_index("s") * NC + lax.axis_index("c")
    J0 = wid * JCHUNK

    lane = lax.iota(jnp.int32, LANES)
    base = lane * D_IN
    zeros = jnp.zeros((LANES,), jnp.float32)

    pltpu.sync_copy(wt_in_hbm.at[:, pl.ds(J0, JCHUNK)], wv)
    pltpu.sync_copy(mt_in_hbm.at[:, pl.ds(J0, JCHUNK)], mv)

    def one_sub(sb, carry):
        c0 = sb * NCOL

        def zero_step(i, c):
            off = i * (LANES * 8)
            for u in range(8):
                blk[pl.ds(off + u * LANES, LANES)] = zeros
            return c

        lax.fori_loop(0, BLK_WORDS // (LANES * 8), zero_step, 0)

        def scat_step(k, c):
            idx = base + mv[k, pl.ds(c0, NCOL)]
            plsc.addupdate_scatter(blk, (idx,), wv[k, pl.ds(c0, NCOL)])
            return c

        lax.fori_loop(0, FAN_IN, scat_step, 0)

        pltpu.sync_copy(blk, dense_hbm.at[pl.ds((J0 + c0) * D_IN, BLK_WORDS)])
        return carry

    lax.fori_loop(0, SUBBLKS, one_sub, 0)


def _densify(condensed_weight, input_mask):
    wt_in = condensed_weight.T
    mt_in = input_mask.T
    mesh = plsc.VectorSubcoreMesh(core_axis_name="c", subcore_axis_name="s")
    dense_flat = pl.kernel(
        _densify_body,
        out_type=jax.ShapeDtypeStruct((D_OUT * D_IN,), jnp.float32),
        mesh=mesh,
        compiler_params=pltpu.CompilerParams(needs_layout_passes=False),
        scratch_types=[
            pltpu.VMEM((BLK_WORDS,), jnp.float32),
            pltpu.VMEM((FAN_IN, JCHUNK), jnp.float32),
            pltpu.VMEM((FAN_IN, JCHUNK), jnp.int32),
        ],
    )(wt_in, mt_in)
    return dense_flat.reshape(D_OUT, D_IN)


M_BLK = 512
N_BLK = 512


def _matmul_body(x_ref, w_ref, b_ref, o_ref):
    acc = lax.dot_general(
        x_ref[...],
        w_ref[...],
        dimension_numbers=(((1,), (1,)), ((), ())),
        preferred_element_type=jnp.float32,
        precision=lax.Precision.DEFAULT,
    )
    o_ref[...] = acc + b_ref[...][None, :]


def _matmul(x, wt, bias):
    grid = (N_TOK // M_BLK, D_OUT // N_BLK)
    return pl.pallas_call(
        _matmul_body,
        grid=grid,
        in_specs=[
            pl.BlockSpec((M_BLK, D_IN), lambda m, n: (m, 0)),
            pl.BlockSpec((N_BLK, D_IN), lambda m, n: (n, 0)),
            pl.BlockSpec((N_BLK,), lambda m, n: (n,)),
        ],
        out_specs=pl.BlockSpec((M_BLK, N_BLK), lambda m, n: (m, n)),
        out_shape=jax.ShapeDtypeStruct((N_TOK, D_OUT), jnp.float32),
    )(x, wt, bias)


@jax.jit
def kernel(input, condensed_weight, input_mask, bias):
    wt = _densify(condensed_weight, input_mask)
    return _matmul(input, wt, bias)


# speedup vs baseline: 22.0582x; 1.1165x over previous
import functools

import jax
import jax.numpy as jnp
from jax import lax
from jax.experimental import pallas as pl
from jax.experimental.pallas import tpu as pltpu
from jax.experimental.pallas import tpu_sc as plsc

N_TOK = 2048
D_IN = 4096
D_OUT = 4096
FAN_IN = 128

NC = 2
NS = 16
NW = NC * NS
LANES = 16

NCOL = 16
JCHUNK = D_OUT // NW
SUBBLKS = JCHUNK // NCOL
BLK_WORDS = NCOL * D_IN


def _densify_body(wt_in_hbm, mt_in_hbm, dense_hbm, blk, wv, mv):
    wid = lax.axis_index("s") * NC + lax.axis_index("c")
    J0 = wid * JCHUNK

    lane = lax.iota(jnp.int32, LANES)
    base = lane * D_IN
    zeros = jnp.zeros((LANES,), jnp.float32)

    pltpu.sync_copy(wt_in_hbm.at[:, pl.ds(J0, JCHUNK)], wv)
    pltpu.sync_copy(mt_in_hbm.at[:, pl.ds(J0, JCHUNK)], mv)

    def one_sub(sb, carry):
        c0 = sb * NCOL

        def zero_step(i, c):
            off = i * (LANES * 8)
            for u in range(8):
                blk[pl.ds(off + u * LANES, LANES)] = zeros
            return c

        lax.fori_loop(0, BLK_WORDS // (LANES * 8), zero_step, 0)

        def scat_step(k, c):
            idx = base + mv[k, pl.ds(c0, NCOL)]
            plsc.addupdate_scatter(blk, (idx,), wv[k, pl.ds(c0, NCOL)])
            return c

        lax.fori_loop(0, FAN_IN, scat_step, 0)

        pltpu.sync_copy(blk, dense_hbm.at[pl.ds((J0 + c0) * D_IN, BLK_WORDS)])
        return carry

    lax.fori_loop(0, SUBBLKS, one_sub, 0)


def _densify(condensed_weight, input_mask):
    wt_in = condensed_weight.T
    mt_in = input_mask.T
    mesh = plsc.VectorSubcoreMesh(core_axis_name="c", subcore_axis_name="s")
    dense_flat = pl.kernel(
        _densify_body,
        out_type=jax.ShapeDtypeStruct((D_OUT * D_IN,), jnp.float32),
        mesh=mesh,
        compiler_params=pltpu.CompilerParams(needs_layout_passes=False),
        scratch_types=[
            pltpu.VMEM((BLK_WORDS,), jnp.float32),
            pltpu.VMEM((FAN_IN, JCHUNK), jnp.float32),
            pltpu.VMEM((FAN_IN, JCHUNK), jnp.int32),
        ],
    )(wt_in, mt_in)
    return dense_flat.reshape(D_OUT, D_IN)


M_BLK = 2048
N_BLK = 256


def _matmul_body(x_ref, w_ref, b_ref, o_ref):
    acc = lax.dot_general(
        x_ref[...],
        w_ref[...],
        dimension_numbers=(((1,), (1,)), ((), ())),
        preferred_element_type=jnp.float32,
        precision=lax.Precision.DEFAULT,
    )
    o_ref[...] = acc + b_ref[...][None, :]


def _matmul(x, wt, bias):
    grid = (D_OUT // N_BLK,)
    return pl.pallas_call(
        _matmul_body,
        grid=grid,
        in_specs=[
            pl.BlockSpec((M_BLK, D_IN), lambda n: (0, 0)),
            pl.BlockSpec((N_BLK, D_IN), lambda n: (n, 0)),
            pl.BlockSpec((N_BLK,), lambda n: (n,)),
        ],
        out_specs=pl.BlockSpec((M_BLK, N_BLK), lambda n: (0, n)),
        out_shape=jax.ShapeDtypeStruct((N_TOK, D_OUT), jnp.float32),
    )(x, wt, bias)


@jax.jit
def kernel(input, condensed_weight, input_mask, bias):
    wt = _densify(condensed_weight, input_mask)
    return _matmul(input, wt, bias)


# speedup vs baseline: 22.9647x; 1.0411x over previous
import functools

import jax
import jax.numpy as jnp
from jax import lax
from jax.experimental import pallas as pl
from jax.experimental.pallas import tpu as pltpu
from jax.experimental.pallas import tpu_sc as plsc

N_TOK = 2048
D_IN = 4096
D_OUT = 4096
FAN_IN = 128

NC = 2
NS = 16
NW = NC * NS
LANES = 16

NCOL = 16
JCHUNK = D_OUT // NW
SUBBLKS = JCHUNK // NCOL
BLK_WORDS = NCOL * D_IN


def _densify_body(wt_in_hbm, mt_in_hbm, dense_hbm, blk, wv, mv):
    wid = lax.axis_index("s") * NC + lax.axis_index("c")
    J0 = wid * JCHUNK

    lane = lax.iota(jnp.int32, LANES)
    base = lane * D_IN
    zeros = jnp.zeros((LANES,), jnp.float32)

    pltpu.sync_copy(wt_in_hbm.at[:, pl.ds(J0, JCHUNK)], wv)
    pltpu.sync_copy(mt_in_hbm.at[:, pl.ds(J0, JCHUNK)], mv)

    def zero_step(i, c):
        off = i * (LANES * 8)
        for u in range(8):
            blk[pl.ds(off + u * LANES, LANES)] = zeros
        return c

    lax.fori_loop(0, BLK_WORDS // (LANES * 8), zero_step, 0)

    def one_sub(sb, carry):
        c0 = sb * NCOL

        def scat_step(k, c):
            idx = base + mv[k, pl.ds(c0, NCOL)]
            plsc.addupdate_scatter(blk, (idx,), wv[k, pl.ds(c0, NCOL)])
            return c

        lax.fori_loop(0, FAN_IN, scat_step, 0)

        pltpu.sync_copy(blk, dense_hbm.at[pl.ds((J0 + c0) * D_IN, BLK_WORDS)])

        def unscat_step(k, c):
            idx = base + mv[k, pl.ds(c0, NCOL)]
            plsc.store_scatter(blk, (idx,), zeros)
            return c

        lax.fori_loop(0, FAN_IN, unscat_step, 0)
        return carry

    lax.fori_loop(0, SUBBLKS, one_sub, 0)


def _densify(condensed_weight, input_mask):
    wt_in = condensed_weight.T
    mt_in = input_mask.T
    mesh = plsc.VectorSubcoreMesh(core_axis_name="c", subcore_axis_name="s")
    dense_flat = pl.kernel(
        _densify_body,
        out_type=jax.ShapeDtypeStruct((D_OUT * D_IN,), jnp.float32),
        mesh=mesh,
        compiler_params=pltpu.CompilerParams(needs_layout_passes=False),
        scratch_types=[
            pltpu.VMEM((BLK_WORDS,), jnp.float32),
            pltpu.VMEM((FAN_IN, JCHUNK), jnp.float32),
            pltpu.VMEM((FAN_IN, JCHUNK), jnp.int32),
        ],
    )(wt_in, mt_in)
    return dense_flat.reshape(D_OUT, D_IN)


M_BLK = 2048
N_BLK = 256


def _matmul_body(x_ref, w_ref, b_ref, o_ref):
    acc = lax.dot_general(
        x_ref[...],
        w_ref[...],
        dimension_numbers=(((1,), (1,)), ((), ())),
        preferred_element_type=jnp.float32,
        precision=lax.Precision.DEFAULT,
    )
    o_ref[...] = acc + b_ref[...][None, :]


def _matmul(x, wt, bias):
    grid = (D_OUT // N_BLK,)
    return pl.pallas_call(
        _matmul_body,
        grid=grid,
        in_specs=[
            pl.BlockSpec((M_BLK, D_IN), lambda n: (0, 0)),
            pl.BlockSpec((N_BLK, D_IN), lambda n: (n, 0)),
            pl.BlockSpec((N_BLK,), lambda n: (n,)),
        ],
        out_specs=pl.BlockSpec((M_BLK, N_BLK), lambda n: (0, n)),
        out_shape=jax.ShapeDtypeStruct((N_TOK, D_OUT), jnp.float32),
    )(x, wt, bias)


@jax.jit
def kernel(input, condensed_weight, input_mask, bias):
    wt = _densify(condensed_weight, input_mask)
    return _matmul(input, wt, bias)


# speedup vs baseline: 32.5790x; 1.4187x over previous
import functools

import jax
import jax.numpy as jnp
from jax import lax
from jax.experimental import pallas as pl
from jax.experimental.pallas import tpu as pltpu
from jax.experimental.pallas import tpu_sc as plsc

N_TOK = 2048
D_IN = 4096
D_OUT = 4096
FAN_IN = 128

NC = 2
NS = 16
NW = NC * NS
LANES = 16

NCOL = 16
JCHUNK = D_OUT // NW
SUBBLKS = JCHUNK // NCOL
BLK_WORDS = NCOL * D_IN


def _densify_body(wt_in_hbm, mt_in_hbm, dense_hbm, blk, wv, mv):
    wid = lax.axis_index("s") * NC + lax.axis_index("c")
    J0 = wid * JCHUNK

    lane = lax.iota(jnp.int32, LANES)
    zeros = jnp.zeros((LANES,), jnp.float32)

    pltpu.sync_copy(wt_in_hbm.at[:, pl.ds(J0, JCHUNK)], wv)
    pltpu.sync_copy(mt_in_hbm.at[:, pl.ds(J0, JCHUNK)], mv)

    def zero_row(c, carry):
        def zero_step(i, cc):
            off = i * (LANES * 8)
            for u in range(8):
                blk[c, pl.ds(off + u * LANES, LANES)] = zeros
            return cc

        lax.fori_loop(0, D_IN // (LANES * 8), zero_step, 0)
        return carry

    lax.fori_loop(0, NCOL, zero_row, 0)

    def one_sub(sb, carry):
        c0 = sb * NCOL

        def scat_step(k, c):
            idx = mv[k, pl.ds(c0, NCOL)]
            plsc.addupdate_scatter(blk, (lane, idx), wv[k, pl.ds(c0, NCOL)])
            return c

        lax.fori_loop(0, FAN_IN, scat_step, 0)

        pltpu.sync_copy(blk, dense_hbm.at[pl.ds(J0 + c0, NCOL), :])

        def unscat_step(k, c):
            idx = mv[k, pl.ds(c0, NCOL)]
            plsc.store_scatter(blk, (lane, idx), zeros)
            return c

        lax.fori_loop(0, FAN_IN, unscat_step, 0)
        return carry

    lax.fori_loop(0, SUBBLKS, one_sub, 0)


def _densify(condensed_weight, input_mask):
    wt_in = condensed_weight.T
    mt_in = input_mask.T
    mesh = plsc.VectorSubcoreMesh(core_axis_name="c", subcore_axis_name="s")
    return pl.kernel(
        _densify_body,
        out_type=jax.ShapeDtypeStruct((D_OUT, D_IN), jnp.float32),
        mesh=mesh,
        compiler_params=pltpu.CompilerParams(needs_layout_passes=False),
        scratch_types=[
            pltpu.VMEM((NCOL, D_IN), jnp.float32),
            pltpu.VMEM((FAN_IN, JCHUNK), jnp.float32),
            pltpu.VMEM((FAN_IN, JCHUNK), jnp.int32),
        ],
    )(wt_in, mt_in)


M_BLK = 2048
N_BLK = 256


def _matmul_body(x_ref, w_ref, b_ref, o_ref):
    acc = lax.dot_general(
        x_ref[...],
        w_ref[...],
        dimension_numbers=(((1,), (1,)), ((), ())),
        preferred_element_type=jnp.float32,
        precision=lax.Precision.DEFAULT,
    )
    o_ref[...] = acc + b_ref[...][None, :]


def _matmul(x, wt, bias):
    grid = (D_OUT // N_BLK,)
    return pl.pallas_call(
        _matmul_body,
        grid=grid,
        in_specs=[
            pl.BlockSpec((M_BLK, D_IN), lambda n: (0, 0)),
            pl.BlockSpec((N_BLK, D_IN), lambda n: (n, 0)),
            pl.BlockSpec((N_BLK,), lambda n: (n,)),
        ],
        out_specs=pl.BlockSpec((M_BLK, N_BLK), lambda n: (0, n)),
        out_shape=jax.ShapeDtypeStruct((N_TOK, D_OUT), jnp.float32),
    )(x, wt, bias)


@jax.jit
def kernel(input, condensed_weight, input_mask, bias):
    wt = _densify(condensed_weight, input_mask)
    return _matmul(input, wt, bias)


# speedup vs baseline: 32.7612x; 1.0056x over previous
import functools

import jax
import jax.numpy as jnp
from jax import lax
from jax.experimental import pallas as pl
from jax.experimental.pallas import tpu as pltpu
from jax.experimental.pallas import tpu_sc as plsc

N_TOK = 2048
D_IN = 4096
D_OUT = 4096
FAN_IN = 128

NC = 2
NS = 16
NW = NC * NS
LANES = 16

CHUNKS = 2
CHUNK_J = D_OUT // CHUNKS
JC = CHUNK_J // NW
STAGE = 2 * JC
NCOL = 16
SUBBLKS = JC // NCOL


def _densify_body(chunk_off, wt_in_hbm, mt_in_hbm, dense_hbm, blk, wv, mv):
    wid = lax.axis_index("s") * NC + lax.axis_index("c")
    stage_j0 = chunk_off + (wid // 2) * STAGE
    col0 = (wid % 2) * JC

    pltpu.sync_copy(wt_in_hbm.at[:, pl.ds(stage_j0, STAGE)], wv)
    pltpu.sync_copy(mt_in_hbm.at[:, pl.ds(stage_j0, STAGE)], mv)

    lane = lax.iota(jnp.int32, LANES)
    zeros = jnp.zeros((LANES,), jnp.float32)

    def zero_row(c, carry):
        def zero_step(i, cc):
            off = i * (LANES * 8)
            for u in range(8):
                blk[c, pl.ds(off + u * LANES, LANES)] = zeros
            return cc

        lax.fori_loop(0, D_IN // (LANES * 8), zero_step, 0)
        return carry

    lax.fori_loop(0, NCOL, zero_row, 0)

    def one_sub(sb, carry):
        c0 = col0 + sb * NCOL

        def scat_step(k, c):
            idx = mv[k, pl.ds(c0, NCOL)]
            plsc.addupdate_scatter(blk, (lane, idx), wv[k, pl.ds(c0, NCOL)])
            return c

        lax.fori_loop(0, FAN_IN, scat_step, 0)

        row0 = wid * JC + sb * NCOL
        pltpu.sync_copy(blk, dense_hbm.at[pl.ds(row0, NCOL), :])

        def unscat_step(k, c):
            idx = mv[k, pl.ds(c0, NCOL)]
            plsc.store_scatter(blk, (lane, idx), zeros)
            return c

        lax.fori_loop(0, FAN_IN, unscat_step, 0)
        return carry

    lax.fori_loop(0, SUBBLKS, one_sub, 0)


def _densify_chunk(wt_in, mt_in, chunk):
    mesh = plsc.VectorSubcoreMesh(core_axis_name="c", subcore_axis_name="s")
    return pl.kernel(
        functools.partial(_densify_body, chunk * CHUNK_J),
        out_type=jax.ShapeDtypeStruct((CHUNK_J, D_IN), jnp.float32),
        mesh=mesh,
        compiler_params=pltpu.CompilerParams(needs_layout_passes=False),
        scratch_types=[
            pltpu.VMEM((NCOL, D_IN), jnp.float32),
            pltpu.VMEM((FAN_IN, STAGE), jnp.float32),
            pltpu.VMEM((FAN_IN, STAGE), jnp.int32),
        ],
        name=f"densify_chunk{chunk}",
    )(wt_in, mt_in)


M_BLK = 2048
N_BLK = 256
N_GRID = CHUNK_J // N_BLK


def _matmul_first_body(x_ref, w_ref, b_ref, o_ref):
    acc = lax.dot_general(
        x_ref[...],
        w_ref[...],
        dimension_numbers=(((1,), (1,)), ((), ())),
        preferred_element_type=jnp.float32,
        precision=lax.Precision.DEFAULT,
    )
    o_ref[...] = acc + b_ref[...][None, :]


def _matmul_rest_body(x_ref, w_ref, b_ref, prev_ref, o_ref):
    del prev_ref
    _matmul_first_body(x_ref, w_ref, b_ref, o_ref)


def _matmul_chunk(x, wt_chunk, bias_chunk, chunk, prev_out=None):
    x_spec = pl.BlockSpec((M_BLK, D_IN), lambda n: (0, 0))
    w_spec = pl.BlockSpec((N_BLK, D_IN), lambda n: (n, 0))
    b_spec = pl.BlockSpec((N_BLK,), lambda n: (n,))
    col0 = chunk * N_GRID
    out_spec = pl.BlockSpec((M_BLK, N_BLK), lambda n: (0, n + col0))
    out_shape = jax.ShapeDtypeStruct((N_TOK, D_OUT), jnp.float32)
    if prev_out is None:
        return pl.pallas_call(
            _matmul_first_body,
            grid=(N_GRID,),
            in_specs=[x_spec, w_spec, b_spec],
            out_specs=out_spec,
            out_shape=out_shape,
        )(x, wt_chunk, bias_chunk)
    return pl.pallas_call(
        _matmul_rest_body,
        grid=(N_GRID,),
        in_specs=[x_spec, w_spec, b_spec,
                  pl.BlockSpec(memory_space=pl.ANY)],
        out_specs=out_spec,
        out_shape=out_shape,
        input_output_aliases={3: 0},
    )(x, wt_chunk, bias_chunk, prev_out)


@jax.jit
def kernel(input, condensed_weight, input_mask, bias):
    wt_in = condensed_weight.T
    mt_in = input_mask.T

    out = None
    for chunk in range(CHUNKS):
        dense = _densify_chunk(wt_in, mt_in, chunk)
        bias_c = lax.slice(bias, (chunk * CHUNK_J,), ((chunk + 1) * CHUNK_J,))
        out = _matmul_chunk(input, dense, bias_c, chunk, out)
    return out
